# baseline (speedup 1.0000x reference)
import functools

import jax
import jax.numpy as jnp
from jax import lax
from jax.experimental import pallas as pl
from jax.experimental.pallas import tpu as pltpu
from jax.experimental.pallas import tpu_sc as plsc

VOCAB = 1000000
EMSIZE = 64
B_TOTAL = 4096 * 200
NW = 32
IDX_PER_W = B_TOTAL // NW
C = 640
G = IDX_PER_W // C

_mesh = plsc.VectorSubcoreMesh(core_axis_name="c", subcore_axis_name="s")


@functools.partial(
    pl.kernel,
    mesh=_mesh,
    out_type=jax.ShapeDtypeStruct((B_TOTAL, EMSIZE), jnp.float32),
    scratch_types=[
        pltpu.VMEM((IDX_PER_W,), jnp.int32),
        pltpu.VMEM((2, C, EMSIZE), jnp.float32),
        pltpu.SemaphoreType.DMA((2,)),
        pltpu.SemaphoreType.DMA((2,)),
        pltpu.SemaphoreType.DMA,
    ],
    compiler_params=pltpu.CompilerParams(use_tc_tiling_on_sc=False),
)
def _gather_kernel(idx_hbm, table_hbm, out_hbm, idx_v, rows_v,
                   gat_sem, out_sem, idx_sem):
    wid = lax.axis_index("s") * 2 + lax.axis_index("c")
    base = wid * IDX_PER_W

    pltpu.async_copy(
        idx_hbm.at[pl.ds(base, IDX_PER_W)], idx_v, idx_sem).wait()

    def fire(g, b):
        pltpu.async_copy(
            table_hbm.at[idx_v.at[pl.ds(g * C, C)]],
            rows_v.at[b], gat_sem.at[b])

    def gat_wait(b):
        pltpu.make_async_copy(
            table_hbm.at[idx_v.at[pl.ds(0, C)]],
            rows_v.at[b], gat_sem.at[b]).wait()

    def wb(g, b):
        o0 = base + g * C
        return pltpu.make_async_copy(
            rows_v.at[b], out_hbm.at[pl.ds(o0, C)], out_sem.at[b])

    def step(g, b, *, first, last):
        if not last:
            if not first:
                wb(g - 1, 1 - b).wait()
            fire(g + 1, 1 - b)
        gat_wait(b)
        wb(g, b).start()

    fire(0, 0)
    step(0, 0, first=True, last=False)

    @pl.loop(0, (G - 2) // 2)
    def _steady(i):
        g0 = 1 + 2 * i
        step(g0, 1, first=False, last=False)
        step(g0 + 1, 0, first=False, last=False)

    step(G - 1, 1, first=False, last=True)
    wb(G - 2, 0).wait()
    wb(G - 1, 1).wait()


def kernel(input_variable, weight):
    idx = input_variable.astype(jnp.int32).reshape(B_TOTAL)
    out = _gather_kernel(idx, weight)
    return out.reshape(input_variable.shape[0], input_variable.shape[1], EMSIZE)


# speedup vs baseline: 1.0029x; 1.0029x over previous
import functools

import jax
import jax.numpy as jnp
from jax import lax
from jax.experimental import pallas as pl
from jax.experimental.pallas import tpu as pltpu
from jax.experimental.pallas import tpu_sc as plsc

VOCAB = 1000000
EMSIZE = 64
B_TOTAL = 4096 * 200
NW = 32
IDX_PER_W = B_TOTAL // NW
C = 320
VL = 16
G = IDX_PER_W // C

_mesh = plsc.VectorSubcoreMesh(core_axis_name="c", subcore_axis_name="s")


@functools.partial(
    pl.kernel,
    mesh=_mesh,
    out_type=jax.ShapeDtypeStruct((B_TOTAL, EMSIZE), jnp.float32),
    scratch_types=[
        pltpu.VMEM((IDX_PER_W,), jnp.int32),
        pltpu.VMEM((2, C, EMSIZE), jnp.float32),
        pltpu.SemaphoreType.DMA((2,)),
        pltpu.SemaphoreType.DMA((2,)),
        pltpu.SemaphoreType.DMA,
    ],
    compiler_params=pltpu.CompilerParams(use_tc_tiling_on_sc=False),
)
def _gather_kernel(idx_hbm, table_hbm, out_hbm, idx_v, rows_v,
                   gat_sem, out_sem, idx_sem):
    wid = lax.axis_index("s") * 2 + lax.axis_index("c")
    base = wid * IDX_PER_W

    pltpu.async_copy(
        idx_hbm.at[pl.ds(base, IDX_PER_W)], idx_v, idx_sem).wait()

    def fire(g, b):
        for j in range(C // VL):
            v = idx_v[pl.ds(g * C + j * VL, VL)]
            pltpu.async_copy(
                table_hbm.at[v],
                rows_v.at[b, pl.ds(j * VL, VL)], gat_sem.at[b])

    def gat_wait(b):
        pltpu.make_async_copy(
            table_hbm.at[idx_v.at[pl.ds(0, C)]],
            rows_v.at[b], gat_sem.at[b]).wait()

    def wb(g, b):
        o0 = base + g * C
        return pltpu.make_async_copy(
            rows_v.at[b], out_hbm.at[pl.ds(o0, C)], out_sem.at[b])

    def step(g, b, *, first, last):
        if not last:
            if not first:
                wb(g - 1, 1 - b).wait()
            fire(g + 1, 1 - b)
        gat_wait(b)
        wb(g, b).start()

    fire(0, 0)
    step(0, 0, first=True, last=False)

    @pl.loop(0, (G - 2) // 2)
    def _steady(i):
        g0 = 1 + 2 * i
        step(g0, 1, first=False, last=False)
        step(g0 + 1, 0, first=False, last=False)

    step(G - 1, 1, first=False, last=True)
    wb(G - 2, 0).wait()
    wb(G - 1, 1).wait()


def kernel(input_variable, weight):
    idx = input_variable.astype(jnp.int32).reshape(B_TOTAL)
    out = _gather_kernel(idx, weight)
    return out.reshape(input_variable.shape[0], input_variable.shape[1], EMSIZE)
